# baseline (speedup 1.0000x reference)
import functools

import jax
import jax.numpy as jnp
from jax import lax
from jax.experimental import pallas as pl
from jax.experimental.pallas import tpu as pltpu
from jax.experimental.pallas import tpu_sc as plsc

F = 26
E = 16
BB = 128
_PAIRS = [(i, j) for i in range(F) for j in range(i + 1, F)]
P = len(_PAIRS)

_RVB = 8192


_FG = 13


def _repack_fm(fme):
    V = fme.shape[1]
    nblk = -(-V // _RVB)
    vg = nblk * (_RVB // 8)
    rows = _FG * E

    def body(in_ref, out_ref):
        x = in_ref[...]
        xT = lax.dot_general(x, jnp.eye(rows, dtype=jnp.float32),
                             (((0,), (0,)), ((), ())),
                             preferred_element_type=jnp.float32)
        x3 = xT.reshape(_RVB // 8, 8, rows)
        for f in range(_FG):
            for r in range(8):
                out_ref[f, :, r * E:(r + 1) * E] = x3[:, r, f * E:(f + 1) * E]

    return pl.pallas_call(
        body,
        grid=(F // _FG, nblk),
        in_specs=[pl.BlockSpec((rows, _RVB), lambda i, j: (i, j))],
        out_specs=pl.BlockSpec((_FG, _RVB // 8, 128), lambda i, j: (i, j, 0)),
        out_shape=jax.ShapeDtypeStruct((F, vg, 128), jnp.float32),
        compiler_params=pltpu.CompilerParams(
            dimension_semantics=("parallel", "parallel")),
    )(fme)


def _sc_gather(gidxT, roffT, lgidxT, lloffT, fmc, linc):
    B = gidxT.shape[1]
    info = plsc.get_sparse_core_info()
    nw = info.num_cores * info.num_subcores
    bpw = B // nw
    ngrp = bpw // 16
    mesh = plsc.VectorSubcoreMesh(core_axis_name="c", subcore_axis_name="s")

    @functools.partial(
        pl.kernel,
        mesh=mesh,
        out_type=(
            jax.ShapeDtypeStruct((F, E, B), jnp.float32),
            jax.ShapeDtypeStruct((F, B), jnp.float32),
        ),
        scratch_types=[
            pltpu.VMEM((F, bpw), jnp.int32),
            pltpu.VMEM((F, bpw), jnp.int32),
            pltpu.VMEM((F, bpw), jnp.int32),
            pltpu.VMEM((F, bpw), jnp.int32),
            pltpu.VMEM((2, bpw, 128), jnp.float32),
            pltpu.VMEM((F, E, bpw), jnp.float32),
            pltpu.VMEM((F, bpw), jnp.float32),
            pltpu.SemaphoreType.DMA,
            pltpu.SemaphoreType.DMA,
        ],
        compiler_params=pltpu.CompilerParams(needs_layout_passes=False),
    )
    def k(gidx_hbm, roff_hbm, lgidx_hbm, lloff_hbm, fmc_hbm, linc_hbm,
          fmT_out, linT_out, gidx_v, roff_v, lgidx_v, lloff_v, rec_v,
          fmT_v, linT_v, sem_f, sem_l):
        wid = lax.axis_index("s") * info.num_cores + lax.axis_index("c")
        base = wid * bpw
        pltpu.sync_copy(gidx_hbm.at[:, pl.ds(base, bpw)], gidx_v)
        pltpu.sync_copy(roff_hbm.at[:, pl.ds(base, bpw)], roff_v)
        pltpu.sync_copy(lgidx_hbm.at[:, pl.ds(base, bpw)], lgidx_v)
        pltpu.sync_copy(lloff_hbm.at[:, pl.ds(base, bpw)], lloff_v)
        iota = lax.iota(jnp.int32, 16)

        def fire_fm(f, buf):
            pltpu.async_copy(fmc_hbm.at[f].at[gidx_v.at[f]],
                             rec_v.at[buf], sem_f)

        def drain_fm(f, buf):
            pltpu.make_async_copy(fmc_hbm.at[f].at[gidx_v.at[f]],
                                  rec_v.at[buf], sem_f).wait()

        fire_fm(0, 0)

        @pl.loop(0, F)
        def _per_field(f):
            p = lax.rem(f, 2)

            @pl.when(f + 1 < F)
            def _start_next():
                fire_fm(f + 1, 1 - p)

            drain_fm(f, p)
            for g in range(ngrp):
                ridx = g * 16 + iota
                roff = roff_v[f, pl.ds(g * 16, 16)]
                for e in range(E):
                    vals = plsc.load_gather(rec_v.at[p], [ridx, roff + e])
                    fmT_v[f, e, pl.ds(g * 16, 16)] = vals

        def fire_lin(f, buf):
            pltpu.async_copy(linc_hbm.at[f].at[lgidx_v.at[f]],
                             rec_v.at[buf], sem_l)

        def drain_lin(f, buf):
            pltpu.make_async_copy(linc_hbm.at[f].at[lgidx_v.at[f]],
                                  rec_v.at[buf], sem_l).wait()

        fire_lin(0, 0)

        @pl.loop(0, F)
        def _per_lin_field(f):
            p = lax.rem(f, 2)

            @pl.when(f + 1 < F)
            def _start_next():
                fire_lin(f + 1, 1 - p)

            drain_lin(f, p)
            for g in range(ngrp):
                ridx = g * 16 + iota
                loff = lloff_v[f, pl.ds(g * 16, 16)]
                lvals = plsc.load_gather(rec_v.at[p], [ridx, loff])
                linT_v[f, pl.ds(g * 16, 16)] = lvals

        pltpu.sync_copy(fmT_v, fmT_out.at[:, :, pl.ds(base, bpw)])
        pltpu.sync_copy(linT_v, linT_out.at[:, pl.ds(base, bpw)])

    return k(gidxT, roffT, lgidxT, lloffT, fmc, linc)


def _tc_body(fmT_ref, dense_ref, linT_ref, W1e_ref, W1d_ref, b1_ref, W2_ref,
             b2_ref, Wl_ref, bl_ref, aW_ref, ab_ref, ah_ref, pp_ref,
             out_ref, pw_ref):
    f32 = jnp.float32
    dn = (((0,), (0,)), ((), ()))
    fmT = fmT_ref[...]

    x1 = lax.dot_general(W1e_ref[...], fmT, dn, preferred_element_type=f32)
    x1 = x1 + lax.dot_general(W1d_ref[...], dense_ref[...],
                              (((0,), (1,)), ((), ())), preferred_element_type=f32)
    h1 = jnp.maximum(x1 + b1_ref[...], 0.0)
    h2 = jnp.maximum(
        lax.dot_general(W2_ref[...], h1, dn, preferred_element_type=f32)
        + b2_ref[...], 0.0)
    lat = jnp.maximum(
        lax.dot_general(Wl_ref[...], h2, dn, preferred_element_type=f32)
        + bl_ref[...], 0.0)

    fms = [fmT[i * E:(i + 1) * E, :] for i in range(F)]
    for p, (i, j) in enumerate(_PAIRS):
        pw_ref[:, p * BB:(p + 1) * BB] = fms[i] * fms[j]
    PW = pw_ref[...]

    h_att = jnp.maximum(
        lax.dot_general(aW_ref[...], PW, dn, preferred_element_type=f32)
        + ab_ref[...], 0.0)
    att = lax.dot_general(ah_ref[...], h_att, dn,
                          preferred_element_type=f32)

    m = att[:, 0:BB]
    for p in range(1, P):
        m = jnp.maximum(m, att[:, p * BB:(p + 1) * BB])
    s = jnp.zeros((1, BB), f32)
    accs = [jnp.zeros((E, BB), f32) for _ in range(4)]
    for p in range(P):
        e_p = jnp.exp(att[:, p * BB:(p + 1) * BB] - m)
        s = s + e_p
        accs[p % 4] = accs[p % 4] + e_p * PW[:, p * BB:(p + 1) * BB]
    afm = ((accs[0] + accs[1]) + (accs[2] + accs[3])) / s

    pred = lax.dot_general(pp_ref[...], afm + lat, dn,
                           preferred_element_type=f32)
    linp = jnp.sum(linT_ref[...], axis=0, keepdims=True)
    z = pred + linp
    out_ref[...] = 1.0 / (1.0 + jnp.exp(-z))


def _tc_forward(fmT, dense, linT, W1e, W1d, b1c, W2, b2c, Wl, blc,
                aW, abc, ah, pp):
    B = fmT.shape[1]
    grid = (B // BB,)
    f32 = jnp.float32
    full = lambda shape: pl.BlockSpec(shape, lambda i: (0, 0))
    return pl.pallas_call(
        _tc_body,
        grid=grid,
        in_specs=[
            pl.BlockSpec((F * E, BB), lambda i: (0, i)),
            pl.BlockSpec((BB, dense.shape[1]), lambda i: (i, 0)),
            pl.BlockSpec((F, BB), lambda i: (0, i)),
            full(W1e.shape), full(W1d.shape), full(b1c.shape),
            full(W2.shape), full(b2c.shape), full(Wl.shape), full(blc.shape),
            full(aW.shape), full(abc.shape), full(ah.shape), full(pp.shape),
        ],
        out_specs=pl.BlockSpec((1, BB), lambda i: (0, i)),
        out_shape=jax.ShapeDtypeStruct((1, B), f32),
        scratch_shapes=[pltpu.VMEM((E, P * BB), f32)],
        compiler_params=pltpu.CompilerParams(
            dimension_semantics=("parallel",)),
    )(fmT, dense, linT, W1e, W1d, b1c, W2, b2c, Wl, blc, aW, abc, ah, pp)


def kernel(sparse_indices, dense_features, fm_tables, lin_tables, W1, b1,
           W2, b2, Wl, bl, attn_W, attn_b, attn_h, proj_p):
    B = sparse_indices.shape[0]
    V = fm_tables.shape[1]

    idxT = sparse_indices.T.astype(jnp.int32)
    gidxT = idxT >> 3
    roffT = (idxT & 7) * E
    lgidxT = idxT >> 7
    lloffT = idxT & 127

    fme = fm_tables.transpose(0, 2, 1).reshape(F * E, V)
    fmc = _repack_fm(fme)

    vl = -(-V // 128) * 128
    linc = jnp.pad(lin_tables.reshape(F, V), ((0, 0), (0, vl - V))).reshape(
        F, vl // 128, 128)

    fmT3, linT = _sc_gather(gidxT, roffT, lgidxT, lloffT, fmc, linc)
    fmT = fmT3.reshape(F * E, B)

    d_in = F * E
    W1e = W1[:d_in]
    W1d = W1[d_in:]
    out = _tc_forward(
        fmT, dense_features.astype(jnp.float32), linT,
        W1e, W1d, b1.reshape(-1, 1), W2, b2.reshape(-1, 1),
        Wl, bl.reshape(-1, 1), attn_W, attn_b.reshape(-1, 1), attn_h, proj_p)
    return out.reshape(B, 1)


# speedup vs baseline: 1.0236x; 1.0236x over previous
import functools

import jax
import jax.numpy as jnp
from jax import lax
from jax.experimental import pallas as pl
from jax.experimental.pallas import tpu as pltpu
from jax.experimental.pallas import tpu_sc as plsc

F = 26
E = 16
BB = 128
_PAIRS = [(i, j) for i in range(F) for j in range(i + 1, F)]
P = len(_PAIRS)

_RVB = 4096


_FG = 13


def _repack_fm(fme):
    V = fme.shape[1]
    nblk = -(-V // _RVB)
    vg = nblk * (_RVB // 8)
    rows = _FG * E

    def body(in_ref, out_ref):
        x = in_ref[...]
        xT = lax.dot_general(x, jnp.eye(rows, dtype=jnp.float32),
                             (((0,), (0,)), ((), ())),
                             preferred_element_type=jnp.float32)
        x3 = xT.reshape(_RVB // 8, 8, rows)
        for f in range(_FG):
            for r in range(8):
                out_ref[f, :, r * E:(r + 1) * E] = x3[:, r, f * E:(f + 1) * E]

    return pl.pallas_call(
        body,
        grid=(F // _FG, nblk),
        in_specs=[pl.BlockSpec((rows, _RVB), lambda i, j: (i, j))],
        out_specs=pl.BlockSpec((_FG, _RVB // 8, 128), lambda i, j: (i, j, 0)),
        out_shape=jax.ShapeDtypeStruct((F, vg, 128), jnp.float32),
        compiler_params=pltpu.CompilerParams(
            dimension_semantics=("parallel", "parallel")),
    )(fme)


def _sc_gather(gidxT, roffT, lgidxT, lloffT, fmc, linc):
    B = gidxT.shape[1]
    info = plsc.get_sparse_core_info()
    nw = info.num_cores * info.num_subcores
    bpw = B // nw
    ngrp = bpw // 16
    mesh = plsc.VectorSubcoreMesh(core_axis_name="c", subcore_axis_name="s")

    @functools.partial(
        pl.kernel,
        mesh=mesh,
        out_type=(
            jax.ShapeDtypeStruct((F, E, B), jnp.float32),
            jax.ShapeDtypeStruct((F, B), jnp.float32),
        ),
        scratch_types=[
            pltpu.VMEM((F, bpw), jnp.int32),
            pltpu.VMEM((F, bpw), jnp.int32),
            pltpu.VMEM((F, bpw), jnp.int32),
            pltpu.VMEM((F, bpw), jnp.int32),
            pltpu.VMEM((2, bpw, 128), jnp.float32),
            pltpu.VMEM((F, E, bpw), jnp.float32),
            pltpu.VMEM((F, bpw), jnp.float32),
            pltpu.SemaphoreType.DMA,
            pltpu.SemaphoreType.DMA,
        ],
        compiler_params=pltpu.CompilerParams(needs_layout_passes=False),
    )
    def k(gidx_hbm, roff_hbm, lgidx_hbm, lloff_hbm, fmc_hbm, linc_hbm,
          fmT_out, linT_out, gidx_v, roff_v, lgidx_v, lloff_v, rec_v,
          fmT_v, linT_v, sem_f, sem_l):
        wid = lax.axis_index("s") * info.num_cores + lax.axis_index("c")
        base = wid * bpw
        pltpu.sync_copy(gidx_hbm.at[:, pl.ds(base, bpw)], gidx_v)
        pltpu.sync_copy(roff_hbm.at[:, pl.ds(base, bpw)], roff_v)
        pltpu.sync_copy(lgidx_hbm.at[:, pl.ds(base, bpw)], lgidx_v)
        pltpu.sync_copy(lloff_hbm.at[:, pl.ds(base, bpw)], lloff_v)
        iota = lax.iota(jnp.int32, 16)

        def fire_fm(f, buf):
            pltpu.async_copy(fmc_hbm.at[f].at[gidx_v.at[f]],
                             rec_v.at[buf], sem_f)

        def drain_fm(f, buf):
            pltpu.make_async_copy(fmc_hbm.at[f].at[gidx_v.at[f]],
                                  rec_v.at[buf], sem_f).wait()

        fire_fm(0, 0)

        @pl.loop(0, F)
        def _per_field(f):
            p = lax.rem(f, 2)

            @pl.when(f + 1 < F)
            def _start_next():
                fire_fm(f + 1, 1 - p)

            drain_fm(f, p)
            for g in range(ngrp):
                ridx = g * 16 + iota
                roff = roff_v[f, pl.ds(g * 16, 16)]
                for e in range(E):
                    vals = plsc.load_gather(rec_v.at[p], [ridx, roff + e])
                    fmT_v[f, e, pl.ds(g * 16, 16)] = vals

        def fire_lin(f, buf):
            pltpu.async_copy(linc_hbm.at[f].at[lgidx_v.at[f]],
                             rec_v.at[buf], sem_l)

        def drain_lin(f, buf):
            pltpu.make_async_copy(linc_hbm.at[f].at[lgidx_v.at[f]],
                                  rec_v.at[buf], sem_l).wait()

        fire_lin(0, 0)

        @pl.loop(0, F)
        def _per_lin_field(f):
            p = lax.rem(f, 2)

            @pl.when(f + 1 < F)
            def _start_next():
                fire_lin(f + 1, 1 - p)

            drain_lin(f, p)
            for g in range(ngrp):
                ridx = g * 16 + iota
                loff = lloff_v[f, pl.ds(g * 16, 16)]
                lvals = plsc.load_gather(rec_v.at[p], [ridx, loff])
                linT_v[f, pl.ds(g * 16, 16)] = lvals

        pltpu.sync_copy(fmT_v, fmT_out.at[:, :, pl.ds(base, bpw)])
        pltpu.sync_copy(linT_v, linT_out.at[:, pl.ds(base, bpw)])

    return k(gidxT, roffT, lgidxT, lloffT, fmc, linc)


def _tc_body(fmT_ref, dense_ref, linT_ref, W1e_ref, W1d_ref, b1_ref, W2_ref,
             b2_ref, Wl_ref, bl_ref, aW_ref, ab_ref, ah_ref, pp_ref,
             out_ref, pw_ref):
    f32 = jnp.float32
    dn = (((0,), (0,)), ((), ()))
    fmT = fmT_ref[...]

    x1 = lax.dot_general(W1e_ref[...], fmT, dn, preferred_element_type=f32)
    x1 = x1 + lax.dot_general(W1d_ref[...], dense_ref[...],
                              (((0,), (1,)), ((), ())), preferred_element_type=f32)
    h1 = jnp.maximum(x1 + b1_ref[...], 0.0)
    h2 = jnp.maximum(
        lax.dot_general(W2_ref[...], h1, dn, preferred_element_type=f32)
        + b2_ref[...], 0.0)
    lat = jnp.maximum(
        lax.dot_general(Wl_ref[...], h2, dn, preferred_element_type=f32)
        + bl_ref[...], 0.0)

    fms = [fmT[i * E:(i + 1) * E, :] for i in range(F)]
    for p, (i, j) in enumerate(_PAIRS):
        pw_ref[:, p * BB:(p + 1) * BB] = fms[i] * fms[j]
    PW = pw_ref[...]

    h_att = jnp.maximum(
        lax.dot_general(aW_ref[...], PW, dn, preferred_element_type=f32)
        + ab_ref[...], 0.0)
    att = lax.dot_general(ah_ref[...], h_att, dn,
                          preferred_element_type=f32)

    m = att[:, 0:BB]
    for p in range(1, P):
        m = jnp.maximum(m, att[:, p * BB:(p + 1) * BB])
    s = jnp.zeros((1, BB), f32)
    accs = [jnp.zeros((E, BB), f32) for _ in range(4)]
    for p in range(P):
        e_p = jnp.exp(att[:, p * BB:(p + 1) * BB] - m)
        s = s + e_p
        accs[p % 4] = accs[p % 4] + e_p * PW[:, p * BB:(p + 1) * BB]
    afm = ((accs[0] + accs[1]) + (accs[2] + accs[3])) / s

    pred = lax.dot_general(pp_ref[...], afm + lat, dn,
                           preferred_element_type=f32)
    linp = jnp.sum(linT_ref[...], axis=0, keepdims=True)
    z = pred + linp
    out_ref[...] = 1.0 / (1.0 + jnp.exp(-z))


def _tc_forward(fmT, dense, linT, W1e, W1d, b1c, W2, b2c, Wl, blc,
                aW, abc, ah, pp):
    B = fmT.shape[1]
    grid = (B // BB,)
    f32 = jnp.float32
    full = lambda shape: pl.BlockSpec(shape, lambda i: (0, 0))
    return pl.pallas_call(
        _tc_body,
        grid=grid,
        in_specs=[
            pl.BlockSpec((F * E, BB), lambda i: (0, i)),
            pl.BlockSpec((BB, dense.shape[1]), lambda i: (i, 0)),
            pl.BlockSpec((F, BB), lambda i: (0, i)),
            full(W1e.shape), full(W1d.shape), full(b1c.shape),
            full(W2.shape), full(b2c.shape), full(Wl.shape), full(blc.shape),
            full(aW.shape), full(abc.shape), full(ah.shape), full(pp.shape),
        ],
        out_specs=pl.BlockSpec((1, BB), lambda i: (0, i)),
        out_shape=jax.ShapeDtypeStruct((1, B), f32),
        scratch_shapes=[pltpu.VMEM((E, P * BB), f32)],
        compiler_params=pltpu.CompilerParams(
            dimension_semantics=("parallel",)),
    )(fmT, dense, linT, W1e, W1d, b1c, W2, b2c, Wl, blc, aW, abc, ah, pp)


def kernel(sparse_indices, dense_features, fm_tables, lin_tables, W1, b1,
           W2, b2, Wl, bl, attn_W, attn_b, attn_h, proj_p):
    B = sparse_indices.shape[0]
    V = fm_tables.shape[1]

    idxT = sparse_indices.T.astype(jnp.int32)
    gidxT = idxT >> 3
    roffT = (idxT & 7) * E
    lgidxT = idxT >> 7
    lloffT = idxT & 127

    fme = fm_tables.transpose(0, 2, 1).reshape(F * E, V)
    fmc = _repack_fm(fme)

    vl = -(-V // 128) * 128
    linc = jnp.pad(lin_tables.reshape(F, V), ((0, 0), (0, vl - V))).reshape(
        F, vl // 128, 128)

    fmT3, linT = _sc_gather(gidxT, roffT, lgidxT, lloffT, fmc, linc)
    fmT = fmT3.reshape(F * E, B)

    d_in = F * E
    W1e = W1[:d_in]
    W1d = W1[d_in:]
    out = _tc_forward(
        fmT, dense_features.astype(jnp.float32), linT,
        W1e, W1d, b1.reshape(-1, 1), W2, b2.reshape(-1, 1),
        Wl, bl.reshape(-1, 1), attn_W, attn_b.reshape(-1, 1), attn_h, proj_p)
    return out.reshape(B, 1)
